# baseline (speedup 1.0000x reference)
import dataclasses
import functools

import jax
import jax.numpy as jnp
from jax import lax
from jax.experimental import pallas as pl
from jax.experimental.pallas import tpu as pltpu
from jax.experimental.pallas import tpu_sc as plsc

N = 10000
NPAD = 10240
HID = 32
OUT = 4
E = 320000
NTILES = 32
EPT = E // NTILES
CHUNK = 80
NCHUNK = EPT // CHUNK
VPC = CHUNK // 16
RPS = NPAD // 16
SUBC = 5
CHUNKL = SUBC * CHUNK
NL = EPT // CHUNKL
GPC = CHUNKL // 16
F32 = jnp.float32


def _pro_vals(xin, w, a_s, a_d):
    h = jnp.dot(xin, w, preferred_element_type=F32)
    A = jnp.sum(h * a_s, axis=1, keepdims=True)
    B = jnp.sum(h * a_d, axis=1, keepdims=True)
    gm = jnp.max(A) + jnp.max(B)
    g = jnp.maximum(gm, 0.2 * gm)
    eb = A + B
    es = jnp.exp(jnp.maximum(eb, 0.2 * eb) - g)
    return h, A, B, g, es


def _pro_body(x_ref, w_ref, as_ref, ad_ref, h_ref, a_ref, b_ref, g_ref, es_ref):
    h, A, B, g, es = _pro_vals(x_ref[...], w_ref[...], as_ref[...], ad_ref[...])
    h_ref[...] = h
    a_ref[...] = A
    b_ref[...] = B
    g_ref[...] = jnp.broadcast_to(g, (1, 1))
    es_ref[...] = es


def _combine_body(rawp_ref, denp_ref, es_ref, h_ref, bias_ref, o_ref):
    raw = rawp_ref[0, 0:N, :] + rawp_ref[1, 0:N, :] + es_ref[...] * h_ref[...]
    den = denp_ref[0, 0:N, :] + denp_ref[1, 0:N, :] + es_ref[...]
    o_ref[...] = jnp.maximum(raw / den + bias_ref[...], 0.0)


def _head_body(x_ref, wp_ref, bp_ref, o_ref):
    o_ref[...] = jnp.dot(x_ref[...], wp_ref[...],
                         preferred_element_type=F32) + bp_ref[...]


_pro_out = [jax.ShapeDtypeStruct((N, HID), F32),
            jax.ShapeDtypeStruct((N, 1), F32),
            jax.ShapeDtypeStruct((N, 1), F32),
            jax.ShapeDtypeStruct((1, 1), F32),
            jax.ShapeDtypeStruct((N, 1), F32)]

_prologue = pl.pallas_call(_pro_body, out_shape=_pro_out)

_tc_combine = pl.pallas_call(
    _combine_body, out_shape=jax.ShapeDtypeStruct((N, HID), F32))

_head = pl.pallas_call(_head_body, out_shape=jax.ShapeDtypeStruct((N, OUT), F32))


_sc_mesh = plsc.VectorSubcoreMesh(core_axis_name="c", subcore_axis_name="s")

_GATHER_DN = lax.GatherDimensionNumbers(
    offset_dims=(), collapsed_slice_dims=(0,), start_index_map=(0,))


def _lane_broadcast(vec, t):
    idx = jnp.full((16, 1), t, jnp.int32)
    return lax.gather(vec, idx, _GATHER_DN, (1,),
                      mode=lax.GatherScatterMode.PROMISE_IN_BOUNDS)

_sc_params = pltpu.CompilerParams(needs_layout_passes=False,
                                  use_tc_tiling_on_sc=False)


@functools.partial(
    pl.kernel,
    compiler_params=_sc_params,
    out_type=[jax.ShapeDtypeStruct((2, NPAD, HID), F32),
              jax.ShapeDtypeStruct((2, NPAD), F32)],
    mesh=_sc_mesh,
    scratch_types=[
        pltpu.VMEM_SHARED((NPAD, HID), F32),
        pltpu.VMEM_SHARED((NPAD,), F32),
        pltpu.VMEM((N,), F32),
        pltpu.VMEM((N,), F32),
        pltpu.VMEM((16,), F32),
        pltpu.VMEM((NCHUNK, CHUNK), jnp.int32),
        pltpu.VMEM((NCHUNK, CHUNK), jnp.int32),
        pltpu.VMEM((SUBC, CHUNK), F32),
        pltpu.VMEM((SUBC, CHUNK), F32),
        pltpu.VMEM((CHUNKL, HID), F32),
        pltpu.VMEM((CHUNKL, HID), F32),
        pltpu.VMEM((CHUNKL, HID), F32),
        pltpu.VMEM((CHUNKL, HID), F32),
        pltpu.VMEM((RPS,), F32),
        pltpu.SemaphoreType.DMA,
        pltpu.SemaphoreType.DMA,
        pltpu.SemaphoreType.DMA,
        pltpu.SemaphoreType.DMA,
    ],
)
def _edge_kernel(src_hbm, dst_hbm, a_hbm, b_hbm, g_hbm, h_hbm,
                 rawp_hbm, denp_hbm,
                 acc_rows, acc_den, a_v, b_v, g_v, src_v, dst_v,
                 ee_v0, ee_v1, rows_in0, rows_in1, rows_sc0, rows_sc1,
                 stage_den, sem_g0, sem_g1, sem_s0, sem_s1):
    c = lax.axis_index("c")
    s = lax.axis_index("s")
    wid = c * 16 + s
    zero16 = jnp.zeros((16,), F32)
    HRPS = RPS // 2

    @pl.loop(0, HRPS)
    def _(i):
        rows_in0[i, 0:16] = zero16
        rows_in0[i, 16:32] = zero16

    @pl.loop(0, RPS // 16)
    def _(i):
        stage_den[pl.ds(i * 16, 16)] = zero16

    pltpu.sync_copy(rows_in0.at[pl.ds(0, HRPS)],
                    acc_rows.at[pl.ds(s * RPS, HRPS)])
    pltpu.sync_copy(rows_in0.at[pl.ds(0, HRPS)],
                    acc_rows.at[pl.ds(s * RPS + HRPS, HRPS)])
    pltpu.sync_copy(stage_den, acc_den.at[pl.ds(s * RPS, RPS)])

    pltpu.sync_copy(a_hbm, a_v)
    pltpu.sync_copy(b_hbm, b_v)
    pltpu.sync_copy(g_hbm, g_v)
    pltpu.sync_copy(src_hbm.at[wid], src_v)
    pltpu.sync_copy(dst_hbm.at[wid], dst_v)
    plsc.subcore_barrier()

    g = g_v[...]
    rows_in = [rows_in0, rows_in1]
    rows_sc = [rows_sc0, rows_sc1]
    ee_v = [ee_v0, ee_v1]
    sem_g = [sem_g0, sem_g1]
    sem_s = [sem_s0, sem_s1]

    def start_gather(lc, b):
        for i in range(SUBC):
            pltpu.async_copy(h_hbm.at[src_v.at[lc * SUBC + i]],
                             rows_in[b].at[pl.ds(i * CHUNK, CHUNK)], sem_g[b])

    def wait_gather(b):
        for i in range(SUBC):
            pltpu.make_async_copy(h_hbm.at[src_v.at[0]],
                                  rows_in[b].at[pl.ds(i * CHUNK, CHUNK)],
                                  sem_g[b]).wait()

    def process(lc, b):
        rin, rsc, eev = rows_in[b], rows_sc[b], ee_v[b]

        @pl.loop(0, GPC)
        def _(k):
            row = lc * SUBC + k // VPC
            col = (k % VPC) * 16
            av = plsc.load_gather(a_v, [src_v[row, pl.ds(col, 16)]])
            bv = plsc.load_gather(b_v, [dst_v[row, pl.ds(col, 16)]])
            e = av + bv
            e = jnp.maximum(e, 0.2 * e)
            ee = jnp.exp(e - g)
            eev[k // VPC, pl.ds(col, 16)] = ee
            base = k * 16
            for t in range(16):
                i = base + t
                bee = _lane_broadcast(ee, t)
                rsc[i, 0:16] = rin[i, 0:16] * bee
                rsc[i, 16:32] = rin[i, 16:32] * bee

    def start_scatter(lc, b):
        for i in range(SUBC):
            pltpu.async_copy(rows_sc[b].at[pl.ds(i * CHUNK, CHUNK)],
                             acc_rows.at[dst_v.at[lc * SUBC + i]],
                             sem_s[b], add=True)
            pltpu.async_copy(ee_v[b].at[i],
                             acc_den.at[dst_v.at[lc * SUBC + i]],
                             sem_s[b], add=True)

    def wait_scatter(b):
        for i in range(SUBC):
            pltpu.make_async_copy(rows_sc[b].at[pl.ds(i * CHUNK, CHUNK)],
                                  acc_rows.at[dst_v.at[0]], sem_s[b]).wait()
            pltpu.make_async_copy(ee_v[b].at[i], acc_den.at[dst_v.at[0]],
                                  sem_s[b]).wait()

    start_gather(0, 0)
    start_gather(1, 1)
    wait_gather(0)
    process(0, 0)
    start_gather(2, 0)
    start_scatter(0, 0)
    wait_gather(1)
    process(1, 1)
    start_gather(3, 1)
    start_scatter(1, 1)

    @pl.loop(2, NL - 1, step=2)
    def _(j):
        for b in range(2):
            lc = j + b
            wait_gather(b)
            wait_scatter(b)
            process(lc, b)

            @pl.when(lc + 2 < NL)
            def _():
                start_gather(lc + 2, b)

            start_scatter(lc, b)

    wait_gather(0)
    wait_scatter(0)
    process(NL - 1, 0)
    start_scatter(NL - 1, 0)
    wait_scatter(0)
    wait_scatter(1)

    plsc.subcore_barrier()

    for half in range(2):
        off = s * RPS + half * HRPS
        pltpu.sync_copy(acc_rows.at[pl.ds(off, HRPS)],
                        rows_in0.at[pl.ds(0, HRPS)])
        pltpu.sync_copy(rows_in0.at[pl.ds(0, HRPS)],
                        rawp_hbm.at[c, pl.ds(off, HRPS)])
    pltpu.sync_copy(acc_den.at[pl.ds(s * RPS, RPS)], stage_den)
    pltpu.sync_copy(stage_den, denp_hbm.at[c, pl.ds(s * RPS, RPS)])


def kernel(x, edge_index, W1, att1_src, att1_dst, b1, W2, att2_src, att2_dst, b2, Wp, bp):
    ei = edge_index.astype(jnp.int32)
    src3 = ei[0].reshape(NTILES, NCHUNK, CHUNK)
    dst3 = ei[1].reshape(NTILES, NCHUNK, CHUNK)

    h1, A1, B1, g1, Es1 = _prologue(x, W1, att1_src.reshape(1, HID),
                                    att1_dst.reshape(1, HID))
    G1 = jnp.broadcast_to(g1.reshape(()), (16,))
    rawp1, denp1 = _edge_kernel(src3, dst3, A1.reshape(N), B1.reshape(N), G1, h1)
    out1 = _tc_combine(rawp1, denp1.reshape(2, NPAD, 1), Es1, h1,
                       b1.reshape(1, HID))

    h2, A2, B2, g2, Es2 = _prologue(out1, W2, att2_src.reshape(1, HID),
                                    att2_dst.reshape(1, HID))
    G2 = jnp.broadcast_to(g2.reshape(()), (16,))
    rawp2, denp2 = _edge_kernel(src3, dst3, A2.reshape(N), B2.reshape(N), G2, h2)
    out2 = _tc_combine(rawp2, denp2.reshape(2, NPAD, 1), Es2, h2,
                       b2.reshape(1, HID))

    return _head(out2, Wp, bp.reshape(1, OUT))


# speedup vs baseline: 1.5902x; 1.5902x over previous
import dataclasses
import functools

import jax
import jax.numpy as jnp
from jax import lax
from jax.experimental import pallas as pl
from jax.experimental.pallas import tpu as pltpu
from jax.experimental.pallas import tpu_sc as plsc

N = 10000
NPAD = 10240
HID = 32
OUT = 4
E = 320000
NTILES = 32
EPT = E // NTILES
CHUNK = 80
NCHUNK = EPT // CHUNK
VPC = CHUNK // 16
RPS = NPAD // 16
NBUF = 4
F32 = jnp.float32


def _pro_vals(xin, w, a_s, a_d):
    h = jnp.dot(xin, w, preferred_element_type=F32)
    A = jnp.sum(h * a_s, axis=1, keepdims=True)
    B = jnp.sum(h * a_d, axis=1, keepdims=True)
    gm = jnp.max(A) + jnp.max(B)
    g = jnp.maximum(gm, 0.2 * gm)
    eb = A + B
    es = jnp.exp(jnp.maximum(eb, 0.2 * eb) - g)
    return h, A, B, g, es


def _pro_body(x_ref, w_ref, as_ref, ad_ref, h_ref, a_ref, b_ref, g_ref, es_ref):
    h, A, B, g, es = _pro_vals(x_ref[...], w_ref[...], as_ref[...], ad_ref[...])
    h_ref[...] = h
    a_ref[...] = A
    b_ref[...] = B
    g_ref[...] = jnp.broadcast_to(g, (1, 1))
    es_ref[...] = es


def _combine_body(rawp_ref, denp_ref, es_ref, h_ref, bias_ref, o_ref):
    raw = rawp_ref[0, 0:N, :] + rawp_ref[1, 0:N, :] + es_ref[...] * h_ref[...]
    den = denp_ref[0, 0:N, :] + denp_ref[1, 0:N, :] + es_ref[...]
    o_ref[...] = jnp.maximum(raw / den + bias_ref[...], 0.0)


def _head_body(x_ref, wp_ref, bp_ref, o_ref):
    o_ref[...] = jnp.dot(x_ref[...], wp_ref[...],
                         preferred_element_type=F32) + bp_ref[...]


_pro_out = [jax.ShapeDtypeStruct((N, HID), F32),
            jax.ShapeDtypeStruct((N, 1), F32),
            jax.ShapeDtypeStruct((N, 1), F32),
            jax.ShapeDtypeStruct((1, 1), F32),
            jax.ShapeDtypeStruct((N, 1), F32)]

_prologue = pl.pallas_call(_pro_body, out_shape=_pro_out)

_tc_combine = pl.pallas_call(
    _combine_body, out_shape=jax.ShapeDtypeStruct((N, HID), F32))

_head = pl.pallas_call(_head_body, out_shape=jax.ShapeDtypeStruct((N, OUT), F32))


_sc_mesh = plsc.VectorSubcoreMesh(core_axis_name="c", subcore_axis_name="s")

_sc_params = pltpu.CompilerParams(needs_layout_passes=False,
                                  use_tc_tiling_on_sc=False)

_GATHER_DN = lax.GatherDimensionNumbers(
    offset_dims=(), collapsed_slice_dims=(0,), start_index_map=(0,))


def _lane_broadcast(vec, t):
    idx = jnp.full((16, 1), t, jnp.int32)
    return lax.gather(vec, idx, _GATHER_DN, (1,),
                      mode=lax.GatherScatterMode.PROMISE_IN_BOUNDS)


_SC_SCRATCH = (
    [pltpu.VMEM_SHARED((NPAD, HID), F32),
     pltpu.VMEM_SHARED((NPAD,), F32),
     pltpu.VMEM((N,), F32),
     pltpu.VMEM((N,), F32),
     pltpu.VMEM((16,), F32),
     pltpu.VMEM((NCHUNK, CHUNK), jnp.int32),
     pltpu.VMEM((NCHUNK, CHUNK), jnp.int32),
     pltpu.VMEM((RPS // 2, HID), F32),
     pltpu.VMEM((RPS,), F32)]
    + [pltpu.VMEM((CHUNK, HID), F32) for _ in range(NBUF)]
    + [pltpu.VMEM((CHUNK, HID), F32) for _ in range(NBUF)]
    + [pltpu.VMEM((CHUNK,), F32) for _ in range(NBUF)]
    + [pltpu.SemaphoreType.DMA for _ in range(2 * NBUF)]
)


@functools.partial(
    pl.kernel,
    compiler_params=_sc_params,
    out_type=[jax.ShapeDtypeStruct((2, NPAD, HID), F32),
              jax.ShapeDtypeStruct((2, NPAD), F32)],
    mesh=_sc_mesh,
    scratch_types=list(_SC_SCRATCH),
)
def _edge_kernel(src_hbm, dst_hbm, a_hbm, b_hbm, g_hbm, h_hbm,
                 rawp_hbm, denp_hbm,
                 acc_rows, acc_den, a_v, b_v, g_v, src_v, dst_v,
                 stage_rows, stage_den, *bufs):
    rows_in = list(bufs[0:NBUF])
    rows_sc = list(bufs[NBUF:2 * NBUF])
    ee_v = list(bufs[2 * NBUF:3 * NBUF])
    sem_g = list(bufs[3 * NBUF:3 * NBUF + NBUF])
    sem_s = list(bufs[3 * NBUF + NBUF:])

    c = lax.axis_index("c")
    s = lax.axis_index("s")
    wid = c * 16 + s
    zero16 = jnp.zeros((16,), F32)
    HRPS = RPS // 2

    @pl.loop(0, HRPS)
    def _(i):
        stage_rows[i, 0:16] = zero16
        stage_rows[i, 16:32] = zero16

    @pl.loop(0, RPS // 16)
    def _(i):
        stage_den[pl.ds(i * 16, 16)] = zero16

    pltpu.sync_copy(stage_rows, acc_rows.at[pl.ds(s * RPS, HRPS)])
    pltpu.sync_copy(stage_rows, acc_rows.at[pl.ds(s * RPS + HRPS, HRPS)])
    pltpu.sync_copy(stage_den, acc_den.at[pl.ds(s * RPS, RPS)])

    pltpu.sync_copy(a_hbm, a_v)
    pltpu.sync_copy(b_hbm, b_v)
    pltpu.sync_copy(g_hbm, g_v)
    pltpu.sync_copy(src_hbm.at[wid], src_v)
    pltpu.sync_copy(dst_hbm.at[wid], dst_v)
    plsc.subcore_barrier()

    g = g_v[...]

    def start_gather(cur, b):
        pltpu.async_copy(h_hbm.at[src_v.at[cur]], rows_in[b], sem_g[b])

    def wait_gather(b):
        pltpu.make_async_copy(h_hbm.at[src_v.at[0]], rows_in[b], sem_g[b]).wait()

    def process(cur, b):
        rin, rsc, eev = rows_in[b], rows_sc[b], ee_v[b]
        for k in range(VPC):
            sl = pl.ds(k * 16, 16)
            av = plsc.load_gather(a_v, [src_v[cur, sl]])
            bv = plsc.load_gather(b_v, [dst_v[cur, sl]])
            e = av + bv
            e = jnp.maximum(e, 0.2 * e)
            ee = jnp.exp(e - g)
            eev[sl] = ee
            for t in range(16):
                i = k * 16 + t
                bee = _lane_broadcast(ee, t)
                rsc[i, 0:16] = rin[i, 0:16] * bee
                rsc[i, 16:32] = rin[i, 16:32] * bee

    def start_scatter(cur, b):
        pltpu.async_copy(rows_sc[b], acc_rows.at[dst_v.at[cur]], sem_s[b],
                         add=True)
        pltpu.async_copy(ee_v[b], acc_den.at[dst_v.at[cur]], sem_s[b],
                         add=True)

    def wait_scatter(b):
        pltpu.make_async_copy(rows_sc[b], acc_rows.at[dst_v.at[0]],
                              sem_s[b]).wait()
        pltpu.make_async_copy(ee_v[b], acc_den.at[dst_v.at[0]],
                              sem_s[b]).wait()

    for b in range(NBUF):
        start_gather(b, b)

    @pl.loop(0, NCHUNK - 1, step=NBUF)
    def _(j):
        for b in range(NBUF):
            cur = j + b
            wait_gather(b)

            @pl.when(cur >= NBUF)
            def _():
                wait_scatter(b)

            process(cur, b)

            @pl.when(cur + NBUF < NCHUNK)
            def _():
                start_gather(cur + NBUF, b)

            start_scatter(cur, b)

    wait_gather(0)
    wait_scatter(0)
    process(NCHUNK - 1, 0)
    start_scatter(NCHUNK - 1, 0)
    for b in range(NBUF):
        wait_scatter(b)

    plsc.subcore_barrier()

    for half in range(2):
        off = s * RPS + half * HRPS
        pltpu.sync_copy(acc_rows.at[pl.ds(off, HRPS)], stage_rows)
        pltpu.sync_copy(stage_rows, rawp_hbm.at[c, pl.ds(off, HRPS)])
    pltpu.sync_copy(acc_den.at[pl.ds(s * RPS, RPS)], stage_den)
    pltpu.sync_copy(stage_den, denp_hbm.at[c, pl.ds(s * RPS, RPS)])


def kernel(x, edge_index, W1, att1_src, att1_dst, b1, W2, att2_src, att2_dst, b2, Wp, bp):
    ei = edge_index.astype(jnp.int32)
    src3 = ei[0].reshape(NTILES, NCHUNK, CHUNK)
    dst3 = ei[1].reshape(NTILES, NCHUNK, CHUNK)

    h1, A1, B1, g1, Es1 = _prologue(x, W1, att1_src.reshape(1, HID),
                                    att1_dst.reshape(1, HID))
    G1 = jnp.broadcast_to(g1.reshape(()), (16,))
    rawp1, denp1 = _edge_kernel(src3, dst3, A1.reshape(N), B1.reshape(N), G1, h1)
    out1 = _tc_combine(rawp1, denp1.reshape(2, NPAD, 1), Es1, h1,
                       b1.reshape(1, HID))

    h2, A2, B2, g2, Es2 = _prologue(out1, W2, att2_src.reshape(1, HID),
                                    att2_dst.reshape(1, HID))
    G2 = jnp.broadcast_to(g2.reshape(()), (16,))
    rawp2, denp2 = _edge_kernel(src3, dst3, A2.reshape(N), B2.reshape(N), G2, h2)
    out2 = _tc_combine(rawp2, denp2.reshape(2, NPAD, 1), Es2, h2,
                       b2.reshape(1, HID))

    return _head(out2, Wp, bp.reshape(1, OUT))


# speedup vs baseline: 1.8464x; 1.1611x over previous
import dataclasses
import functools

import jax
import jax.numpy as jnp
from jax import lax
from jax.experimental import pallas as pl
from jax.experimental.pallas import tpu as pltpu
from jax.experimental.pallas import tpu_sc as plsc

N = 10000
NPAD = 10240
HID = 32
OUT = 4
E = 320000
NTILES = 32
EPT = E // NTILES
CHUNK = 80
NCHUNK = EPT // CHUNK
VPC = CHUNK // 16
RPS = NPAD // 16
NBUF = 4
F32 = jnp.float32


def _pro_vals(xin, w, a_s, a_d):
    h = jnp.dot(xin, w, preferred_element_type=F32)
    A = jnp.sum(h * a_s, axis=1)
    B = jnp.sum(h * a_d, axis=1)
    gm = jnp.max(A) + jnp.max(B)
    g = jnp.maximum(gm, 0.2 * gm)
    eb = A + B
    es = jnp.exp(jnp.maximum(eb, 0.2 * eb) - g)
    return h, A, B, g, es


def _pro_body(x_ref, w_ref, as_ref, ad_ref, h_ref, a_ref, b_ref, g_ref, es_ref):
    h, A, B, g, es = _pro_vals(x_ref[...], w_ref[...], as_ref[...], ad_ref[...])
    h_ref[...] = h
    a_ref[...] = A
    b_ref[...] = B
    g_ref[...] = jnp.broadcast_to(g, (1, 1))
    es_ref[...] = es


def _combine_body(rawp_ref, denp_ref, es_ref, h_ref, bias_ref, o_ref):
    es = es_ref[...][:, None]
    raw = rawp_ref[0, 0:N, :] + rawp_ref[1, 0:N, :] + es * h_ref[...]
    den = (denp_ref[0, 0:N] + denp_ref[1, 0:N] + es_ref[...])[:, None]
    o_ref[...] = jnp.maximum(raw / den + bias_ref[...], 0.0)


def _head_body(x_ref, wp_ref, bp_ref, o_ref):
    o_ref[...] = jnp.dot(x_ref[...], wp_ref[...],
                         preferred_element_type=F32) + bp_ref[...]


_pro_out = [jax.ShapeDtypeStruct((N, HID), F32),
            jax.ShapeDtypeStruct((N,), F32),
            jax.ShapeDtypeStruct((N,), F32),
            jax.ShapeDtypeStruct((1, 1), F32),
            jax.ShapeDtypeStruct((N,), F32)]

_prologue = pl.pallas_call(_pro_body, out_shape=_pro_out)

_tc_combine = pl.pallas_call(
    _combine_body, out_shape=jax.ShapeDtypeStruct((N, HID), F32))

_head = pl.pallas_call(_head_body, out_shape=jax.ShapeDtypeStruct((N, OUT), F32))


_sc_mesh = plsc.VectorSubcoreMesh(core_axis_name="c", subcore_axis_name="s")

_sc_params = pltpu.CompilerParams(needs_layout_passes=False,
                                  use_tc_tiling_on_sc=False)

_GATHER_DN = lax.GatherDimensionNumbers(
    offset_dims=(), collapsed_slice_dims=(0,), start_index_map=(0,))


def _lane_broadcast(vec, t):
    idx = jnp.full((16, 1), t, jnp.int32)
    return lax.gather(vec, idx, _GATHER_DN, (1,),
                      mode=lax.GatherScatterMode.PROMISE_IN_BOUNDS)


_SC_SCRATCH = (
    [pltpu.VMEM_SHARED((NPAD, HID), F32),
     pltpu.VMEM_SHARED((NPAD,), F32),
     pltpu.VMEM((N,), F32),
     pltpu.VMEM((N,), F32),
     pltpu.VMEM((16,), F32),
     pltpu.VMEM((NCHUNK, CHUNK), jnp.int32),
     pltpu.VMEM((NCHUNK, CHUNK), jnp.int32),
     pltpu.VMEM((RPS // 2, HID), F32),
     pltpu.VMEM((RPS,), F32)]
    + [pltpu.VMEM((CHUNK, HID), F32) for _ in range(NBUF)]
    + [pltpu.VMEM((CHUNK, HID), F32) for _ in range(NBUF)]
    + [pltpu.VMEM((CHUNK,), F32) for _ in range(NBUF)]
    + [pltpu.SemaphoreType.DMA for _ in range(2 * NBUF)]
)


@functools.partial(
    pl.kernel,
    compiler_params=_sc_params,
    out_type=[jax.ShapeDtypeStruct((2, NPAD, HID), F32),
              jax.ShapeDtypeStruct((2, NPAD), F32)],
    mesh=_sc_mesh,
    scratch_types=list(_SC_SCRATCH),
)
def _edge_kernel(src_hbm, dst_hbm, a_hbm, b_hbm, g_hbm, h_hbm,
                 rawp_hbm, denp_hbm,
                 acc_rows, acc_den, a_v, b_v, g_v, src_v, dst_v,
                 stage_rows, stage_den, *bufs):
    rows_in = list(bufs[0:NBUF])
    rows_sc = list(bufs[NBUF:2 * NBUF])
    ee_v = list(bufs[2 * NBUF:3 * NBUF])
    sem_g = list(bufs[3 * NBUF:3 * NBUF + NBUF])
    sem_s = list(bufs[3 * NBUF + NBUF:])

    c = lax.axis_index("c")
    s = lax.axis_index("s")
    wid = c * 16 + s
    zero16 = jnp.zeros((16,), F32)
    HRPS = RPS // 2

    @pl.loop(0, HRPS)
    def _(i):
        stage_rows[i, 0:16] = zero16
        stage_rows[i, 16:32] = zero16

    @pl.loop(0, RPS // 16)
    def _(i):
        stage_den[pl.ds(i * 16, 16)] = zero16

    pltpu.sync_copy(stage_rows, acc_rows.at[pl.ds(s * RPS, HRPS)])
    pltpu.sync_copy(stage_rows, acc_rows.at[pl.ds(s * RPS + HRPS, HRPS)])
    pltpu.sync_copy(stage_den, acc_den.at[pl.ds(s * RPS, RPS)])

    pltpu.sync_copy(a_hbm, a_v)
    pltpu.sync_copy(b_hbm, b_v)
    pltpu.sync_copy(g_hbm, g_v)
    pltpu.sync_copy(src_hbm.at[wid], src_v)
    pltpu.sync_copy(dst_hbm.at[wid], dst_v)
    plsc.subcore_barrier()

    g = g_v[...]

    def start_gather(cur, b):
        pltpu.async_copy(h_hbm.at[src_v.at[cur]], rows_in[b], sem_g[b])

    def wait_gather(b):
        pltpu.make_async_copy(h_hbm.at[src_v.at[0]], rows_in[b], sem_g[b]).wait()

    def process(cur, b):
        rin, rsc, eev = rows_in[b], rows_sc[b], ee_v[b]
        for k in range(VPC):
            sl = pl.ds(k * 16, 16)
            av = plsc.load_gather(a_v, [src_v[cur, sl]])
            bv = plsc.load_gather(b_v, [dst_v[cur, sl]])
            e = av + bv
            e = jnp.maximum(e, 0.2 * e)
            ee = jnp.exp(e - g)
            eev[sl] = ee
            for t in range(16):
                i = k * 16 + t
                bee = _lane_broadcast(ee, t)
                rsc[i, 0:16] = rin[i, 0:16] * bee
                rsc[i, 16:32] = rin[i, 16:32] * bee

    def start_scatter(cur, b):
        pltpu.async_copy(rows_sc[b], acc_rows.at[dst_v.at[cur]], sem_s[b],
                         add=True)
        pltpu.async_copy(ee_v[b], acc_den.at[dst_v.at[cur]], sem_s[b],
                         add=True)

    def wait_scatter(b):
        pltpu.make_async_copy(rows_sc[b], acc_rows.at[dst_v.at[0]],
                              sem_s[b]).wait()
        pltpu.make_async_copy(ee_v[b], acc_den.at[dst_v.at[0]],
                              sem_s[b]).wait()

    for b in range(NBUF):
        start_gather(b, b)

    @pl.loop(0, NCHUNK - 1, step=NBUF)
    def _(j):
        for b in range(NBUF):
            cur = j + b
            wait_gather(b)

            @pl.when(cur >= NBUF)
            def _():
                wait_scatter(b)

            process(cur, b)

            @pl.when(cur + NBUF < NCHUNK)
            def _():
                start_gather(cur + NBUF, b)

            start_scatter(cur, b)

    wait_gather(0)
    wait_scatter(0)
    process(NCHUNK - 1, 0)
    start_scatter(NCHUNK - 1, 0)
    for b in range(NBUF):
        wait_scatter(b)

    plsc.subcore_barrier()

    for half in range(2):
        off = s * RPS + half * HRPS
        pltpu.sync_copy(acc_rows.at[pl.ds(off, HRPS)], stage_rows)
        pltpu.sync_copy(stage_rows, rawp_hbm.at[c, pl.ds(off, HRPS)])
    pltpu.sync_copy(acc_den.at[pl.ds(s * RPS, RPS)], stage_den)
    pltpu.sync_copy(stage_den, denp_hbm.at[c, pl.ds(s * RPS, RPS)])


def kernel(x, edge_index, W1, att1_src, att1_dst, b1, W2, att2_src, att2_dst, b2, Wp, bp):
    ei = edge_index.astype(jnp.int32)
    src3 = ei[0].reshape(NTILES, NCHUNK, CHUNK)
    dst3 = ei[1].reshape(NTILES, NCHUNK, CHUNK)

    h1, A1, B1, g1, Es1 = _prologue(x, W1, att1_src.reshape(1, HID),
                                    att1_dst.reshape(1, HID))
    G1 = jnp.broadcast_to(g1.reshape(()), (16,))
    rawp1, denp1 = _edge_kernel(src3, dst3, A1, B1, G1, h1)
    out1 = _tc_combine(rawp1, denp1, Es1, h1, b1.reshape(1, HID))

    h2, A2, B2, g2, Es2 = _prologue(out1, W2, att2_src.reshape(1, HID),
                                    att2_dst.reshape(1, HID))
    G2 = jnp.broadcast_to(g2.reshape(()), (16,))
    rawp2, denp2 = _edge_kernel(src3, dst3, A2, B2, G2, h2)
    out2 = _tc_combine(rawp2, denp2, Es2, h2, b2.reshape(1, HID))

    return _head(out2, Wp, bp.reshape(1, OUT))
